# Initial kernel scaffold; baseline (speedup 1.0000x reference)
import jax
import jax.numpy as jnp
from jax.experimental import pallas as pl


def kernel(predictions, edge_index, edge_weights):
    raise NotImplementedError("write your pallas kernel here")


# speedup vs baseline: 188.5307x; 188.5307x over previous
import functools

import jax
import jax.numpy as jnp
from jax import lax
from jax.experimental import pallas as pl
from jax.experimental.pallas import tpu as pltpu
from jax.experimental.pallas import tpu_sc as plsc

_LANES = 16
_NW = 32
_LOSS_WEIGHT = 0.1


def _pick_chunk(per_w: int, max_chunk: int = 10016) -> int:
    start = max_chunk - (max_chunk % _LANES)
    for c in range(start, 0, -_LANES):
        if per_w % c == 0:
            return c
    return _LANES


def _make_sc_partial(n_nodes: int, n_edges: int):
    assert n_edges % (_NW * _LANES) == 0, "edge count must tile over 32x16 lanes"
    per_w = n_edges // _NW
    chunk = _pick_chunk(per_w)
    n_chunks = per_w // chunk
    mesh = plsc.VectorSubcoreMesh(core_axis_name="c", subcore_axis_name="s")

    @functools.partial(
        pl.kernel,
        mesh=mesh,
        compiler_params=pltpu.CompilerParams(needs_layout_passes=False),
        out_type=jax.ShapeDtypeStruct((_NW, _LANES), jnp.float32),
        scratch_types=[
            pltpu.VMEM((n_nodes,), jnp.float32),
            pltpu.VMEM((chunk,), jnp.int32),
            pltpu.VMEM((chunk,), jnp.int32),
            pltpu.VMEM((chunk,), jnp.float32),
            pltpu.VMEM((_LANES,), jnp.float32),
        ],
    )
    def sc_partial(pred_hbm, src_hbm, tgt_hbm, w_hbm, out_hbm,
                   pred_v, src_v, tgt_v, w_v, out_v):
        wid = lax.axis_index("s") * 2 + lax.axis_index("c")
        pltpu.sync_copy(pred_hbm, pred_v)
        base = wid * per_w

        def chunk_body(ci, acc):
            off = base + ci * chunk
            pltpu.sync_copy(src_hbm.at[pl.ds(off, chunk)], src_v)
            pltpu.sync_copy(tgt_hbm.at[pl.ds(off, chunk)], tgt_v)
            pltpu.sync_copy(w_hbm.at[pl.ds(off, chunk)], w_v)

            def inner(i, a):
                sl = pl.ds(i * _LANES, _LANES)
                pa = plsc.load_gather(pred_v, [src_v[sl]])
                pb = plsc.load_gather(pred_v, [tgt_v[sl]])
                return a + w_v[sl] * jnp.abs(pa - pb)

            return lax.fori_loop(0, chunk // _LANES, inner, acc)

        acc = lax.fori_loop(0, n_chunks, chunk_body,
                            jnp.zeros((_LANES,), jnp.float32))
        out_v[...] = acc
        pltpu.sync_copy(out_v, out_hbm.at[wid])

    return sc_partial


def kernel(predictions, edge_index, edge_weights):
    n_nodes = predictions.shape[0]
    n_edges = edge_weights.shape[0]
    ei = edge_index.astype(jnp.int32)
    src = ei[0]
    tgt = ei[1]
    partial = _make_sc_partial(n_nodes, n_edges)(
        predictions, src, tgt, edge_weights)
    return jnp.sum(partial) * (_LOSS_WEIGHT / n_edges)


# speedup vs baseline: 223.5835x; 1.1859x over previous
import functools

import jax
import jax.numpy as jnp
from jax import lax
from jax.experimental import pallas as pl
from jax.experimental.pallas import tpu as pltpu
from jax.experimental.pallas import tpu_sc as plsc

_LANES = 16
_NW = 32
_LOSS_WEIGHT = 0.1
_UNROLL = 4


def _pick_chunk(per_w: int, max_chunk: int = 10048) -> int:
    start = max_chunk - (max_chunk % _LANES)
    for c in range(start, 0, -_LANES):
        if per_w % c == 0:
            return c
    return _LANES


def _make_sc_partial(n_nodes: int, n_edges: int):
    assert n_edges % (_NW * _LANES) == 0
    per_w = n_edges // _NW
    chunk = _pick_chunk(per_w)
    n_chunks = per_w // chunk
    mesh = plsc.VectorSubcoreMesh(core_axis_name="c", subcore_axis_name="s")

    @functools.partial(
        pl.kernel,
        mesh=mesh,
        compiler_params=pltpu.CompilerParams(needs_layout_passes=False),
        out_type=jax.ShapeDtypeStruct((_NW, _LANES), jnp.float32),
        scratch_types=[
            pltpu.VMEM((n_nodes,), jnp.float32),
            pltpu.VMEM((chunk,), jnp.int32),
            pltpu.VMEM((chunk,), jnp.int32),
            pltpu.VMEM((chunk,), jnp.int32),
            pltpu.VMEM((chunk,), jnp.int32),
            pltpu.VMEM((chunk,), jnp.float32),
            pltpu.VMEM((chunk,), jnp.float32),
            pltpu.VMEM((_LANES,), jnp.float32),
            pltpu.SemaphoreType.DMA,
            pltpu.SemaphoreType.DMA,
            pltpu.SemaphoreType.DMA,
        ],
    )
    def sc_partial(pred_hbm, src_hbm, tgt_hbm, w_hbm, out_hbm,
                   pred_v, src_v0, src_v1, tgt_v0, tgt_v1, w_v0, w_v1,
                   out_v, psem, sem0, sem1):
        wid = lax.axis_index("s") * 2 + lax.axis_index("c")
        base = wid * per_w
        sems = (sem0, sem1)
        src_v = (src_v0, src_v1)
        tgt_v = (tgt_v0, tgt_v1)
        w_v = (w_v0, w_v1)

        pred_cp = pltpu.async_copy(pred_hbm, pred_v, psem)

        def start(ci):
            b = ci % 2
            off = base + ci * chunk
            return [
                pltpu.async_copy(src_hbm.at[pl.ds(off, chunk)], src_v[b], sems[b]),
                pltpu.async_copy(tgt_hbm.at[pl.ds(off, chunk)], tgt_v[b], sems[b]),
                pltpu.async_copy(w_hbm.at[pl.ds(off, chunk)], w_v[b], sems[b]),
            ]

        inflight = start(0)
        pred_cp.wait()

        zero = jnp.zeros((_LANES,), jnp.float32)
        accs = (zero, zero, zero, zero)

        n_groups = chunk // _LANES
        n_main = (n_groups // _UNROLL) * _UNROLL

        def term(b, g):
            sl = pl.ds(g * _LANES, _LANES)
            return w_v[b][sl] * jnp.abs(
                plsc.load_gather(pred_v, [src_v[b][sl]])
                - plsc.load_gather(pred_v, [tgt_v[b][sl]]))

        def compute_chunk(b, accs_in):
            def body(i, a):
                a0, a1, a2, a3 = a
                return (a0 + term(b, i), a1 + term(b, i + 1),
                        a2 + term(b, i + 2), a3 + term(b, i + 3))

            accs_out = plsc.parallel_loop(
                0, n_main, step=_UNROLL, carry=accs_in)(body)
            for g in range(n_main, n_groups):
                accs_out = (accs_out[0] + term(b, g),) + accs_out[1:]
            return accs_out

        for ci in range(n_chunks):
            b = ci % 2
            if ci + 1 < n_chunks:
                nxt = start(ci + 1)
            for cp in inflight:
                cp.wait()
            if ci + 1 < n_chunks:
                inflight = nxt
            accs = compute_chunk(b, accs)

        a0, a1, a2, a3 = accs
        out_v[...] = (a0 + a1) + (a2 + a3)
        pltpu.sync_copy(out_v, out_hbm.at[wid])

    return sc_partial


def kernel(predictions, edge_index, edge_weights):
    n_nodes = predictions.shape[0]
    n_edges = edge_weights.shape[0]
    ei = edge_index.astype(jnp.int32)
    src = ei[0]
    tgt = ei[1]
    partial = _make_sc_partial(n_nodes, n_edges)(
        predictions, src, tgt, edge_weights)
    return jnp.sum(partial) * (_LOSS_WEIGHT / n_edges)


# speedup vs baseline: 342.9411x; 1.5338x over previous
import functools

import jax
import jax.numpy as jnp
from jax import lax
from jax.experimental import pallas as pl
from jax.experimental.pallas import tpu as pltpu
from jax.experimental.pallas import tpu_sc as plsc

_LANES = 16
_NW = 32
_LOSS_WEIGHT = 0.1
_UNROLL = 4


def _pick_chunk(per_w: int, max_chunk: int = 10048) -> int:
    start = max_chunk - (max_chunk % _LANES)
    for c in range(start, 0, -_LANES):
        if per_w % c == 0:
            return c
    return _LANES


def _make_sc_partial(n_nodes: int, n_edges: int):
    assert n_edges % (_NW * _LANES) == 0
    per_w = n_edges // _NW
    chunk = _pick_chunk(per_w)
    n_chunks = per_w // chunk
    mesh = plsc.VectorSubcoreMesh(core_axis_name="c", subcore_axis_name="s")

    @functools.partial(
        pl.kernel,
        mesh=mesh,
        compiler_params=pltpu.CompilerParams(needs_layout_passes=False),
        out_type=jax.ShapeDtypeStruct((_NW, _LANES), jnp.float32),
        scratch_types=[
            pltpu.VMEM((n_nodes,), jnp.float32),
            pltpu.VMEM((chunk,), jnp.int32),
            pltpu.VMEM((chunk,), jnp.int32),
            pltpu.VMEM((chunk,), jnp.int32),
            pltpu.VMEM((chunk,), jnp.int32),
            pltpu.VMEM((chunk,), jnp.float32),
            pltpu.VMEM((chunk,), jnp.float32),
            pltpu.VMEM((_LANES,), jnp.float32),
            pltpu.SemaphoreType.DMA,
            pltpu.SemaphoreType.DMA,
            pltpu.SemaphoreType.DMA,
        ],
    )
    def sc_partial(pred_hbm, ei_hbm, w_hbm, out_hbm,
                   pred_v, src_v0, src_v1, tgt_v0, tgt_v1, w_v0, w_v1,
                   out_v, psem, sem0, sem1):
        wid = lax.axis_index("s") * 2 + lax.axis_index("c")
        base = wid * per_w
        sems = (sem0, sem1)
        src_v = (src_v0, src_v1)
        tgt_v = (tgt_v0, tgt_v1)
        w_v = (w_v0, w_v1)

        pred_cp = pltpu.async_copy(pred_hbm, pred_v, psem)

        def start(ci):
            b = ci % 2
            off = base + ci * chunk
            return [
                pltpu.async_copy(ei_hbm.at[pl.ds(off, chunk)], src_v[b], sems[b]),
                pltpu.async_copy(ei_hbm.at[pl.ds(n_edges + off, chunk)], tgt_v[b], sems[b]),
                pltpu.async_copy(w_hbm.at[pl.ds(off, chunk)], w_v[b], sems[b]),
            ]

        inflight = start(0)
        pred_cp.wait()

        zero = jnp.zeros((_LANES,), jnp.float32)
        accs = (zero, zero, zero, zero)

        n_groups = chunk // _LANES
        n_main = (n_groups // _UNROLL) * _UNROLL

        def term(b, g):
            sl = pl.ds(g * _LANES, _LANES)
            return w_v[b][sl] * jnp.abs(
                plsc.load_gather(pred_v, [src_v[b][sl]])
                - plsc.load_gather(pred_v, [tgt_v[b][sl]]))

        def compute_chunk(b, accs_in):
            def body(i, a):
                a0, a1, a2, a3 = a
                return (a0 + term(b, i), a1 + term(b, i + 1),
                        a2 + term(b, i + 2), a3 + term(b, i + 3))

            accs_out = plsc.parallel_loop(
                0, n_main, step=_UNROLL, carry=accs_in)(body)
            for g in range(n_main, n_groups):
                accs_out = (accs_out[0] + term(b, g),) + accs_out[1:]
            return accs_out

        for ci in range(n_chunks):
            b = ci % 2
            if ci + 1 < n_chunks:
                nxt = start(ci + 1)
            for cp in inflight:
                cp.wait()
            if ci + 1 < n_chunks:
                inflight = nxt
            accs = compute_chunk(b, accs)

        a0, a1, a2, a3 = accs
        out_v[...] = (a0 + a1) + (a2 + a3)
        pltpu.sync_copy(out_v, out_hbm.at[wid])

    return sc_partial


def kernel(predictions, edge_index, edge_weights):
    n_nodes = predictions.shape[0]
    n_edges = edge_weights.shape[0]
    ei = edge_index.astype(jnp.int32).reshape(-1)
    partial = _make_sc_partial(n_nodes, n_edges)(
        predictions, ei, edge_weights)
    return jnp.sum(partial) * (_LOSS_WEIGHT / n_edges)


# speedup vs baseline: 560.8125x; 1.6353x over previous
import functools

import jax
import jax.numpy as jnp
from jax import lax
from jax.experimental import pallas as pl
from jax.experimental.pallas import tpu as pltpu
from jax.experimental.pallas import tpu_sc as plsc

_LANES = 16
_NW = 32
_BLK = 128
_LOSS_WEIGHT = 0.1
_UNROLL = 4


def _pick_chunk_blocks(per_w_blocks: int, max_blocks: int = 80) -> int:
    for cb in range(max_blocks, 0, -1):
        if per_w_blocks % cb == 0:
            return cb
    return 1


def _make_sc_partial(n_nodes: int, n_edges: int):
    assert n_edges % _BLK == 0
    nb = n_edges // _BLK
    base_blocks = nb // _NW
    n_extra = nb % _NW
    cb = _pick_chunk_blocks(base_blocks)
    chunk = cb * _BLK
    n_chunks = base_blocks // cb
    groups = chunk // _LANES
    assert groups % _UNROLL == 0
    mesh = plsc.VectorSubcoreMesh(core_axis_name="c", subcore_axis_name="s")

    @functools.partial(
        pl.kernel,
        mesh=mesh,
        compiler_params=pltpu.CompilerParams(needs_layout_passes=False),
        out_type=jax.ShapeDtypeStruct((_NW, _LANES), jnp.float32),
        scratch_types=[
            pltpu.VMEM((n_nodes,), jnp.float32),
            pltpu.VMEM((2, chunk), jnp.int32),
            pltpu.VMEM((2, chunk), jnp.int32),
            pltpu.VMEM((chunk,), jnp.float32),
            pltpu.VMEM((chunk,), jnp.float32),
            pltpu.VMEM((2, _BLK), jnp.int32),
            pltpu.VMEM((_BLK,), jnp.float32),
            pltpu.VMEM((_LANES,), jnp.float32),
            pltpu.SemaphoreType.DMA,
            pltpu.SemaphoreType.DMA,
            pltpu.SemaphoreType.DMA,
            pltpu.SemaphoreType.DMA,
        ],
    )
    def sc_partial(pred_hbm, ei_hbm, w_hbm, out_hbm,
                   pred_v, ei_v0, ei_v1, w_v0, w_v1, ei_tail, w_tail,
                   out_v, psem, sem0, sem1, semt):
        wid = lax.axis_index("s") * 2 + lax.axis_index("c")
        sems = (sem0, sem1)
        ei_v = (ei_v0, ei_v1)
        w_v = (w_v0, w_v1)

        start_blk = wid * base_blocks + jnp.minimum(wid, n_extra)
        has_extra = wid < n_extra
        base = start_blk * _BLK

        pred_cp = pltpu.async_copy(pred_hbm, pred_v, psem)

        def start(ci):
            b = ci % 2
            off = pl.multiple_of(base + ci * chunk, _BLK)
            return [
                pltpu.async_copy(ei_hbm.at[:, pl.ds(off, chunk)], ei_v[b], sems[b]),
                pltpu.async_copy(w_hbm.at[pl.ds(off, chunk)], w_v[b], sems[b]),
            ]

        inflight = start(0)
        tail_off = pl.multiple_of(
            jnp.minimum(base + n_chunks * chunk, n_edges - _BLK), _BLK)
        tail_cps = [
            pltpu.async_copy(ei_hbm.at[:, pl.ds(tail_off, _BLK)], ei_tail, semt),
            pltpu.async_copy(w_hbm.at[pl.ds(tail_off, _BLK)], w_tail, semt),
        ]
        pred_cp.wait()

        zero = jnp.zeros((_LANES,), jnp.float32)
        accs = (zero, zero, zero, zero)

        def term(eref, wref, g):
            sl = pl.ds(g * _LANES, _LANES)
            return wref[sl] * jnp.abs(
                plsc.load_gather(pred_v, [eref[0, sl]])
                - plsc.load_gather(pred_v, [eref[1, sl]]))

        def compute_chunk(b, accs_in):
            def body(i, a):
                a0, a1, a2, a3 = a
                return (a0 + term(ei_v[b], w_v[b], i),
                        a1 + term(ei_v[b], w_v[b], i + 1),
                        a2 + term(ei_v[b], w_v[b], i + 2),
                        a3 + term(ei_v[b], w_v[b], i + 3))

            return plsc.parallel_loop(
                0, groups, step=_UNROLL, carry=accs_in)(body)

        for ci in range(n_chunks):
            b = ci % 2
            if ci + 1 < n_chunks:
                nxt = start(ci + 1)
            for cp in inflight:
                cp.wait()
            if ci + 1 < n_chunks:
                inflight = nxt
            accs = compute_chunk(b, accs)

        for cp in tail_cps:
            cp.wait()
        a0, a1, a2, a3 = accs
        tail_sum = zero
        for g in range(_BLK // _LANES):
            tail_sum = tail_sum + term(ei_tail, w_tail, g)
        a0 = a0 + jnp.where(has_extra, tail_sum, zero)

        out_v[...] = (a0 + a1) + (a2 + a3)
        pltpu.sync_copy(out_v, out_hbm.at[wid])

    return sc_partial


def kernel(predictions, edge_index, edge_weights):
    n_nodes = predictions.shape[0]
    n_edges = edge_weights.shape[0]
    ei = edge_index.astype(jnp.int32)
    partial = _make_sc_partial(n_nodes, n_edges)(
        predictions, ei, edge_weights)
    return jnp.sum(partial) * (_LOSS_WEIGHT / n_edges)


# speedup vs baseline: 561.3672x; 1.0010x over previous
import functools

import jax
import jax.numpy as jnp
from jax import lax
from jax.experimental import pallas as pl
from jax.experimental.pallas import tpu as pltpu
from jax.experimental.pallas import tpu_sc as plsc

_LANES = 16
_NW = 32
_BLK = 128
_LOSS_WEIGHT = 0.1
_UNROLL = 4


def _pick_chunk_blocks(per_w_blocks: int, max_blocks: int = 80) -> int:
    for cb in range(max_blocks, 0, -1):
        if per_w_blocks % cb == 0:
            return cb
    return 1


def _make_sc_partial(n_nodes: int, n_edges: int):
    assert n_edges % _BLK == 0
    nb = n_edges // _BLK
    base_blocks = nb // _NW
    n_extra = nb % _NW
    cb = _pick_chunk_blocks(base_blocks)
    chunk = cb * _BLK
    n_chunks = base_blocks // cb
    groups = chunk // _LANES
    assert groups % _UNROLL == 0
    mesh = plsc.VectorSubcoreMesh(core_axis_name="c", subcore_axis_name="s")

    @functools.partial(
        pl.kernel,
        mesh=mesh,
        compiler_params=pltpu.CompilerParams(needs_layout_passes=False),
        out_type=jax.ShapeDtypeStruct((_NW, _LANES), jnp.float32),
        scratch_types=[
            pltpu.VMEM((n_nodes,), jnp.float32),
            pltpu.VMEM((2, chunk), jnp.int32),
            pltpu.VMEM((2, chunk), jnp.int32),
            pltpu.VMEM((chunk,), jnp.float32),
            pltpu.VMEM((chunk,), jnp.float32),
            pltpu.VMEM((2, _BLK), jnp.int32),
            pltpu.VMEM((_BLK,), jnp.float32),
            pltpu.VMEM((_LANES,), jnp.float32),
            pltpu.SemaphoreType.DMA,
            pltpu.SemaphoreType.DMA,
            pltpu.SemaphoreType.DMA,
            pltpu.SemaphoreType.DMA,
        ],
    )
    def sc_partial(pred_hbm, ei_hbm, w_hbm, out_hbm,
                   pred_v, ei_v0, ei_v1, w_v0, w_v1, ei_tail, w_tail,
                   out_v, psem, sem0, sem1, semt):
        wid = lax.axis_index("s") * 2 + lax.axis_index("c")
        sems = (sem0, sem1)
        ei_v = (ei_v0, ei_v1)
        w_v = (w_v0, w_v1)

        start_blk = wid * base_blocks + jnp.minimum(wid, n_extra)
        has_extra = wid < n_extra
        base = start_blk * _BLK

        pred_cp = pltpu.async_copy(pred_hbm, pred_v, psem)

        def start(ci):
            b = ci % 2
            off = pl.multiple_of(base + ci * chunk, _BLK)
            return [
                pltpu.async_copy(ei_hbm.at[:, pl.ds(off, chunk)], ei_v[b], sems[b]),
                pltpu.async_copy(w_hbm.at[pl.ds(off, chunk)], w_v[b], sems[b]),
            ]

        inflight = start(0)
        tail_off = pl.multiple_of(
            jnp.minimum(base + n_chunks * chunk, n_edges - _BLK), _BLK)
        tail_cps = [
            pltpu.async_copy(ei_hbm.at[:, pl.ds(tail_off, _BLK)], ei_tail, semt),
            pltpu.async_copy(w_hbm.at[pl.ds(tail_off, _BLK)], w_tail, semt),
        ]
        pred_cp.wait()

        zero = jnp.zeros((_LANES,), jnp.float32)
        accs = (zero, zero, zero, zero)

        def term(eref, wref, g):
            sl = pl.ds(g * _LANES, _LANES)
            return wref[sl] * jnp.abs(
                plsc.load_gather(pred_v, [eref[0, sl]])
                - plsc.load_gather(pred_v, [eref[1, sl]]))

        def compute_chunk(b, accs_in):
            def body(i, a):
                a0, a1, a2, a3 = a
                return (a0 + term(ei_v[b], w_v[b], i),
                        a1 + term(ei_v[b], w_v[b], i + 1),
                        a2 + term(ei_v[b], w_v[b], i + 2),
                        a3 + term(ei_v[b], w_v[b], i + 3))

            return plsc.parallel_loop(
                0, groups, step=_UNROLL, unroll=2, carry=accs_in)(body)

        for ci in range(n_chunks):
            b = ci % 2
            if ci + 1 < n_chunks:
                nxt = start(ci + 1)
            for cp in inflight:
                cp.wait()
            if ci + 1 < n_chunks:
                inflight = nxt
            accs = compute_chunk(b, accs)

        for cp in tail_cps:
            cp.wait()
        a0, a1, a2, a3 = accs
        tail_sum = zero
        for g in range(_BLK // _LANES):
            tail_sum = tail_sum + term(ei_tail, w_tail, g)
        a0 = a0 + jnp.where(has_extra, tail_sum, zero)

        out_v[...] = (a0 + a1) + (a2 + a3)
        pltpu.sync_copy(out_v, out_hbm.at[wid])

    return sc_partial


def kernel(predictions, edge_index, edge_weights):
    n_nodes = predictions.shape[0]
    n_edges = edge_weights.shape[0]
    ei = edge_index.astype(jnp.int32)
    partial = _make_sc_partial(n_nodes, n_edges)(
        predictions, ei, edge_weights)
    return jnp.sum(partial) * (_LOSS_WEIGHT / n_edges)
